# Initial kernel scaffold; baseline (speedup 1.0000x reference)
import jax
import jax.numpy as jnp
from jax.experimental import pallas as pl


def kernel(x, edge_index, batch, params):
    raise NotImplementedError("write your pallas kernel here")


# speedup vs baseline: 5.6845x; 5.6845x over previous
import functools

import jax
import jax.numpy as jnp
import numpy as np
from jax import lax
from jax.experimental import pallas as pl
from jax.experimental.pallas import tpu as pltpu
from jax.experimental.pallas import tpu_sc as plsc

N = 10000
E = 320000
D = 128
Hd = 128
L = 5
G = 64
T = 12

NC = 2
NS = 16
NW = NC * NS
CH = 128
CPW = -(-E // (NW * CH))
EPAD = NW * CH * CPW
N_ACC = 10112
ZR = N_ACC // NS

_BN_S = 1.0 / np.sqrt(1.0 + 1e-5)


def _agg_body(h_hbm, src_hbm, dst_hbm, zrows_hbm, out_hbm,
              src_v, dst_v, rows_v, acc, sem):
    c = lax.axis_index("c")
    s = lax.axis_index("s")
    wid = s * NC + c

    pltpu.sync_copy(zrows_hbm, acc.at[pl.ds(s * ZR, ZR)])
    plsc.subcore_barrier()

    def chunk(j, carry):
        base = pl.multiple_of((wid * CPW + j) * CH, CH)
        pltpu.sync_copy(src_hbm.at[pl.ds(base, CH)], src_v)
        pltpu.sync_copy(dst_hbm.at[pl.ds(base, CH)], dst_v)
        pltpu.async_copy(h_hbm.at[src_v], rows_v, sem).wait()
        pltpu.sync_copy(rows_v, acc.at[dst_v], add=True)
        return carry

    lax.fori_loop(0, CPW, chunk, 0)
    plsc.subcore_barrier()
    pltpu.sync_copy(acc.at[pl.ds(s * ZR, ZR)],
                    out_hbm.at[c, pl.ds(s * ZR, ZR)])


@functools.cache
def _make_aggregate():
    return pl.kernel(
        _agg_body,
        out_type=jax.ShapeDtypeStruct((NC, N_ACC, D), jnp.float32),
        mesh=plsc.VectorSubcoreMesh(core_axis_name="c", subcore_axis_name="s",
                                    num_cores=NC, num_subcores=NS),
        scratch_types=[
            pltpu.VMEM((CH,), jnp.int32),
            pltpu.VMEM((CH,), jnp.int32),
            pltpu.VMEM((CH, D), jnp.float32),
            pltpu.VMEM_SHARED((N_ACC, D), jnp.float32),
            pltpu.SemaphoreType.DMA,
        ],
    )


def _aggregate(h, src_p, dst_p, zrows):
    return _make_aggregate()(h, src_p, dst_p, zrows)


def _proj_body(x_ref, w_ref, b_ref, batch_ref, h_ref, cnt_ref):
    h = jnp.dot(x_ref[...], w_ref[...], preferred_element_type=jnp.float32)
    h_ref[...] = jnp.maximum(h + b_ref[...], 0.0)
    seg = lax.broadcasted_iota(jnp.int32, (G, N), 0)
    m = (batch_ref[...] == seg).astype(jnp.float32)
    cnt_ref[...] = jnp.broadcast_to(jnp.sum(m, axis=1, keepdims=True),
                                    (G, 128))


def _proj_call(x, w, b, batch2d):
    return pl.pallas_call(
        _proj_body,
        out_shape=[jax.ShapeDtypeStruct((N, Hd), jnp.float32),
                   jax.ShapeDtypeStruct((G, 128), jnp.float32)],
    )(x, w, b, batch2d)


def _layer_body(h_ref, p_ref, eps_ref, w1_ref, b1_ref, w2_ref, b2_ref,
                batch_ref, hout_ref, pool_ref):
    agg = p_ref[0, :N, :] + p_ref[1, :N, :]
    z = (1.0 + eps_ref[...]) * h_ref[...] + agg
    z = jnp.dot(z, w1_ref[...], preferred_element_type=jnp.float32)
    z = jnp.maximum(z + b1_ref[...], 0.0)
    z = jnp.dot(z, w2_ref[...], preferred_element_type=jnp.float32)
    h2 = jnp.maximum(z + b2_ref[...], 0.0)
    hout_ref[...] = h2
    seg = lax.broadcasted_iota(jnp.int32, (G, N), 0)
    m = (batch_ref[...] == seg).astype(jnp.float32)
    pool_ref[...] = jnp.dot(m, h2, preferred_element_type=jnp.float32)


def _layer_call(h, p, eps2d, w1, b1, w2, b2, batch2d):
    return pl.pallas_call(
        _layer_body,
        out_shape=[jax.ShapeDtypeStruct((N, Hd), jnp.float32),
                   jax.ShapeDtypeStruct((G, Hd), jnp.float32)],
    )(h, p, eps2d, w1, b1, w2, b2, batch2d)


def _head_body(pool_ref, cnt_ref, jkw_ref, jkb_ref, w1_ref, b1_ref,
               w2_ref, b2_ref, w3_ref, b3_ref, out_ref):
    cnt = cnt_ref[:, 0:1]
    acc = jnp.zeros((G, Hd), jnp.float32)
    for i in range(L):
        hgi = jnp.dot(pool_ref[i], jkw_ref[i],
                      preferred_element_type=jnp.float32)
        hgi = hgi + cnt * jkb_ref[i]
        acc = acc + jnp.dot(hgi, w1_ref[i],
                            preferred_element_type=jnp.float32)
    hg = jnp.maximum(acc + b1_ref[...], 0.0)
    hg = jnp.dot(hg, w2_ref[...], preferred_element_type=jnp.float32)
    hg = jnp.maximum(hg + b2_ref[...], 0.0)
    out_ref[...] = (jnp.dot(hg, w3_ref[...],
                            preferred_element_type=jnp.float32)
                    + b3_ref[...])


def _head_call(pooled, cnt, jkw, jkb, w1c, b1, w2, b2, w3, b3):
    return pl.pallas_call(
        _head_body,
        out_shape=jax.ShapeDtypeStruct((G, T), jnp.float32),
    )(pooled, cnt, jkw, jkb, w1c, b1, w2, b2, w3, b3)


def _fold_bn(w, b, g, bb):
    s = g * _BN_S
    return w * s[None, :], (b * s + bb)[None, :]


def kernel(x, edge_index, batch, params):
    p = params
    src = edge_index[0]
    dst = edge_index[1]
    npad = EPAD - E
    pad_src = (jnp.arange(npad, dtype=jnp.int32) % N)
    pad_dst = N + (jnp.arange(npad, dtype=jnp.int32) % (N_ACC - N))
    src_p = jnp.concatenate([src, pad_src])
    dst_p = jnp.concatenate([dst, pad_dst])
    zrows = jnp.zeros((ZR, D), jnp.float32)
    batch2d = batch.reshape(1, N)

    inw, inb = _fold_bn(p['inW'], p['inb'], p['ing'], p['inbb'])
    h, cnt = _proj_call(x, inw, inb, batch2d)

    pooled = []
    for i in range(L):
        gl = p['gin'][i]
        w1, b1 = _fold_bn(gl['W1'], gl['b1'], gl['g1'], gl['bb1'])
        w2, b2 = _fold_bn(gl['W2'], gl['b2'], gl['g2'], gl['bb2'])
        eps2d = gl['eps'].reshape(1, 1)
        part = _aggregate(h, src_p, dst_p, zrows)
        h, pool_i = _layer_call(h, part, eps2d, w1, b1, w2, b2, batch2d)
        pooled.append(pool_i)
    pooled = jnp.stack(pooled)

    jkw = jnp.stack([p['jk'][i]['W'] for i in range(L)])
    jkb = jnp.stack([p['jk'][i]['b'].reshape(1, Hd) for i in range(L)])
    po = p['out']
    ow1, ob1 = _fold_bn(po['W1'], po['b1'], po['g1'], po['bb1'])
    ow2, ob2 = _fold_bn(po['W2'], po['b2'], po['g2'], po['bb2'])
    w1c = ow1.reshape(L, Hd, Hd)
    return _head_call(pooled, cnt, jkw, jkb, w1c, ob1,
                      ow2, ob2, po['W3'], po['b3'].reshape(1, T))


# speedup vs baseline: 8.9551x; 1.5754x over previous
import functools

import jax
import jax.numpy as jnp
import numpy as np
from jax import lax
from jax.experimental import pallas as pl
from jax.experimental.pallas import tpu as pltpu
from jax.experimental.pallas import tpu_sc as plsc

N = 10000
E = 320000
D = 128
Hd = 128
L = 5
G = 64
T = 12

NC = 2
NS = 16
NW = NC * NS
CH = 128
CPW = 80
EPAD = NW * CH * CPW
N_ACC = 10112
ZR = N_ACC // NS

_BN_S = 1.0 / np.sqrt(1.0 + 1e-5)


def _agg_body(h_hbm, src_hbm, dst_hbm, zrows_hbm, out_hbm,
              src0, src1, dst0, dst1, rows0, rows1, acc, gsem0, gsem1):
    c = lax.axis_index("c")
    s = lax.axis_index("s")
    wid = s * NC + c
    ebase = wid * (CPW * CH)

    srcs = (src0, src1)
    dsts = (dst0, dst1)
    rows = (rows0, rows1)
    gsems = (gsem0, gsem1)

    def load_idx(j, b):
        base = pl.multiple_of(ebase + j * CH, CH)
        pltpu.sync_copy(src_hbm.at[pl.ds(base, CH)], srcs[b])
        pltpu.sync_copy(dst_hbm.at[pl.ds(base, CH)], dsts[b])

    def gather(b):
        pltpu.async_copy(h_hbm.at[srcs[b]], rows[b], gsems[b])

    pltpu.sync_copy(zrows_hbm, acc.at[pl.ds(s * ZR, ZR)])
    plsc.subcore_barrier()

    load_idx(0, 0)
    gather(0)
    load_idx(1, 1)
    gather(1)

    def outer(i, carry):
        for b in range(2):
            j = i * 2 + b
            pltpu.make_async_copy(h_hbm.at[pl.ds(0, CH)],
                                  rows[b], gsems[b]).wait()
            pltpu.sync_copy(rows[b], acc.at[dsts[b]], add=True)

            @pl.when(j + 2 < CPW)
            def _():
                load_idx(j + 2, b)
                gather(b)
        return carry

    lax.fori_loop(0, CPW // 2, outer, 0)
    plsc.subcore_barrier()
    pltpu.sync_copy(acc.at[pl.ds(s * ZR, ZR)],
                    out_hbm.at[c, pl.ds(s * ZR, ZR)])


@functools.cache
def _make_aggregate():
    return pl.kernel(
        _agg_body,
        out_type=jax.ShapeDtypeStruct((NC, N_ACC, D), jnp.float32),
        mesh=plsc.VectorSubcoreMesh(core_axis_name="c", subcore_axis_name="s",
                                    num_cores=NC, num_subcores=NS),
        scratch_types=[
            pltpu.VMEM((CH,), jnp.int32),
            pltpu.VMEM((CH,), jnp.int32),
            pltpu.VMEM((CH,), jnp.int32),
            pltpu.VMEM((CH,), jnp.int32),
            pltpu.VMEM((CH, D), jnp.float32),
            pltpu.VMEM((CH, D), jnp.float32),
            pltpu.VMEM_SHARED((N_ACC, D), jnp.float32),
            pltpu.SemaphoreType.DMA,
            pltpu.SemaphoreType.DMA,
        ],
    )


def _aggregate(h, src_p, dst_p, zrows):
    return _make_aggregate()(h, src_p, dst_p, zrows)


def _proj_body(x_ref, w_ref, b_ref, batch_ref, h_ref, cnt_ref):
    h = jnp.dot(x_ref[...], w_ref[...], preferred_element_type=jnp.float32)
    h_ref[...] = jnp.maximum(h + b_ref[...], 0.0)
    seg = lax.broadcasted_iota(jnp.int32, (G, N), 0)
    m = (batch_ref[...] == seg).astype(jnp.float32)
    cnt_ref[...] = jnp.broadcast_to(jnp.sum(m, axis=1, keepdims=True),
                                    (G, 128))


def _proj_call(x, w, b, batch2d):
    return pl.pallas_call(
        _proj_body,
        out_shape=[jax.ShapeDtypeStruct((N, Hd), jnp.float32),
                   jax.ShapeDtypeStruct((G, 128), jnp.float32)],
    )(x, w, b, batch2d)


def _layer_body(h_ref, p_ref, eps_ref, w1_ref, b1_ref, w2_ref, b2_ref,
                batch_ref, hout_ref, pool_ref):
    agg = p_ref[0, :N, :] + p_ref[1, :N, :]
    z = (1.0 + eps_ref[...]) * h_ref[...] + agg
    z = jnp.dot(z, w1_ref[...], preferred_element_type=jnp.float32)
    z = jnp.maximum(z + b1_ref[...], 0.0)
    z = jnp.dot(z, w2_ref[...], preferred_element_type=jnp.float32)
    h2 = jnp.maximum(z + b2_ref[...], 0.0)
    hout_ref[...] = h2
    seg = lax.broadcasted_iota(jnp.int32, (G, N), 0)
    m = (batch_ref[...] == seg).astype(jnp.float32)
    pool_ref[...] = jnp.dot(m, h2, preferred_element_type=jnp.float32)


def _layer_call(h, p, eps2d, w1, b1, w2, b2, batch2d):
    return pl.pallas_call(
        _layer_body,
        out_shape=[jax.ShapeDtypeStruct((N, Hd), jnp.float32),
                   jax.ShapeDtypeStruct((G, Hd), jnp.float32)],
    )(h, p, eps2d, w1, b1, w2, b2, batch2d)


def _head_body(pool_ref, cnt_ref, jkw_ref, jkb_ref, w1_ref, b1_ref,
               w2_ref, b2_ref, w3_ref, b3_ref, out_ref):
    cnt = cnt_ref[:, 0:1]
    acc = jnp.zeros((G, Hd), jnp.float32)
    for i in range(L):
        hgi = jnp.dot(pool_ref[i], jkw_ref[i],
                      preferred_element_type=jnp.float32)
        hgi = hgi + cnt * jkb_ref[i]
        acc = acc + jnp.dot(hgi, w1_ref[i],
                            preferred_element_type=jnp.float32)
    hg = jnp.maximum(acc + b1_ref[...], 0.0)
    hg = jnp.dot(hg, w2_ref[...], preferred_element_type=jnp.float32)
    hg = jnp.maximum(hg + b2_ref[...], 0.0)
    out_ref[...] = (jnp.dot(hg, w3_ref[...],
                            preferred_element_type=jnp.float32)
                    + b3_ref[...])


def _head_call(pooled, cnt, jkw, jkb, w1c, b1, w2, b2, w3, b3):
    return pl.pallas_call(
        _head_body,
        out_shape=jax.ShapeDtypeStruct((G, T), jnp.float32),
    )(pooled, cnt, jkw, jkb, w1c, b1, w2, b2, w3, b3)


def _fold_bn(w, b, g, bb):
    s = g * _BN_S
    return w * s[None, :], (b * s + bb)[None, :]


def kernel(x, edge_index, batch, params):
    p = params
    src = edge_index[0]
    dst = edge_index[1]
    npad = EPAD - E
    pad_src = (jnp.arange(npad, dtype=jnp.int32) % N)
    pad_dst = N + (jnp.arange(npad, dtype=jnp.int32) % (N_ACC - N))
    src_p = jnp.concatenate([src, pad_src])
    dst_p = jnp.concatenate([dst, pad_dst])
    zrows = jnp.zeros((ZR, D), jnp.float32)
    batch2d = batch.reshape(1, N)

    inw, inb = _fold_bn(p['inW'], p['inb'], p['ing'], p['inbb'])
    h, cnt = _proj_call(x, inw, inb, batch2d)

    pooled = []
    for i in range(L):
        gl = p['gin'][i]
        w1, b1 = _fold_bn(gl['W1'], gl['b1'], gl['g1'], gl['bb1'])
        w2, b2 = _fold_bn(gl['W2'], gl['b2'], gl['g2'], gl['bb2'])
        eps2d = gl['eps'].reshape(1, 1)
        part = _aggregate(h, src_p, dst_p, zrows)
        h, pool_i = _layer_call(h, part, eps2d, w1, b1, w2, b2, batch2d)
        pooled.append(pool_i)
    pooled = jnp.stack(pooled)

    jkw = jnp.stack([p['jk'][i]['W'] for i in range(L)])
    jkb = jnp.stack([p['jk'][i]['b'].reshape(1, Hd) for i in range(L)])
    po = p['out']
    ow1, ob1 = _fold_bn(po['W1'], po['b1'], po['g1'], po['bb1'])
    ow2, ob2 = _fold_bn(po['W2'], po['b2'], po['g2'], po['bb2'])
    w1c = ow1.reshape(L, Hd, Hd)
    return _head_call(pooled, cnt, jkw, jkb, w1c, ob1,
                      ow2, ob2, po['W3'], po['b3'].reshape(1, T))


# speedup vs baseline: 10.7770x; 1.2035x over previous
import functools

import jax
import jax.numpy as jnp
import numpy as np
from jax import lax
from jax.experimental import pallas as pl
from jax.experimental.pallas import tpu as pltpu
from jax.experimental.pallas import tpu_sc as plsc

N = 10000
E = 320000
D = 128
Hd = 128
L = 5
G = 64
T = 12

NC = 2
NS = 16
NW = NC * NS
CH = 120
CPW = 84
EPAD = NW * CH * CPW
NB = 3
N_ACC = 10112
ZR = N_ACC // NS

_BN_S = 1.0 / np.sqrt(1.0 + 1e-5)


def _agg_body(h_hbm, src_hbm, dst_hbm, zrows_hbm, out_hbm,
              src0, src1, src2, dst0, dst1, dst2, rows0, rows1, rows2,
              acc, gsem0, gsem1, gsem2, ssem0, ssem1, ssem2):
    c = lax.axis_index("c")
    s = lax.axis_index("s")
    wid = s * NC + c
    ebase = wid * (CPW * CH)

    srcs = (src0, src1, src2)
    dsts = (dst0, dst1, dst2)
    rows = (rows0, rows1, rows2)
    gsems = (gsem0, gsem1, gsem2)
    ssems = (ssem0, ssem1, ssem2)

    def load_src(j, b):
        pltpu.sync_copy(src_hbm.at[pl.ds(pl.multiple_of(ebase + j * CH, CH),
                                         CH)], srcs[b])

    def load_dst(j, b):
        pltpu.sync_copy(dst_hbm.at[pl.ds(pl.multiple_of(ebase + j * CH, CH),
                                         CH)], dsts[b])

    def gather(b):
        pltpu.async_copy(h_hbm.at[srcs[b]], rows[b], gsems[b])

    def wait_gather(b):
        pltpu.make_async_copy(h_hbm.at[pl.ds(0, CH)], rows[b],
                              gsems[b]).wait()

    def wait_scatter(b):
        pltpu.make_async_copy(h_hbm.at[pl.ds(0, CH)], rows[b],
                              ssems[b]).wait()

    pltpu.sync_copy(zrows_hbm, acc.at[pl.ds(s * ZR, ZR)])
    plsc.subcore_barrier()

    for b in range(NB):
        load_src(b, b)
        if b < 2:
            load_dst(b, b)
            gather(b)

    def outer(i, carry):
        for b in range(NB):
            j3 = i * NB
            j = j3 + b
            b1 = (b + 2) % NB
            wait_gather(b)
            pltpu.async_copy(rows[b], acc.at[dsts[b]], ssems[b], add=True)

            @pl.when(j > 0)
            def _():
                wait_scatter(b1)

            @pl.when(j + NB < CPW)
            def _():
                load_src(j + NB, b)

            @pl.when(j + 2 < CPW)
            def _():
                load_dst(j + 2, b1)

            @pl.when(j + 2 < CPW)
            def _():
                gather(b1)
        return carry

    lax.fori_loop(0, CPW // NB, outer, 0)
    wait_scatter((CPW - 1) % NB)
    plsc.subcore_barrier()
    pltpu.sync_copy(acc.at[pl.ds(s * ZR, ZR)],
                    out_hbm.at[c, pl.ds(s * ZR, ZR)])


@functools.cache
def _make_aggregate():
    return pl.kernel(
        _agg_body,
        out_type=jax.ShapeDtypeStruct((NC, N_ACC, D), jnp.float32),
        mesh=plsc.VectorSubcoreMesh(core_axis_name="c", subcore_axis_name="s",
                                    num_cores=NC, num_subcores=NS),
        scratch_types=(
            [pltpu.VMEM((CH,), jnp.int32) for _ in range(2 * NB)]
            + [pltpu.VMEM((CH, D), jnp.float32) for _ in range(NB)]
            + [pltpu.VMEM_SHARED((N_ACC, D), jnp.float32)]
            + [pltpu.SemaphoreType.DMA for _ in range(2 * NB)]
        ),
    )


def _aggregate(h, src_p, dst_p, zrows):
    return _make_aggregate()(h, src_p, dst_p, zrows)


def _proj_body(x_ref, w_ref, b_ref, batch_ref, h_ref, cnt_ref):
    h = jnp.dot(x_ref[...], w_ref[...], preferred_element_type=jnp.float32)
    h_ref[...] = jnp.maximum(h + b_ref[...], 0.0)
    seg = lax.broadcasted_iota(jnp.int32, (G, N), 0)
    m = (batch_ref[...] == seg).astype(jnp.float32)
    cnt_ref[...] = jnp.broadcast_to(jnp.sum(m, axis=1, keepdims=True),
                                    (G, 128))


def _proj_call(x, w, b, batch2d):
    return pl.pallas_call(
        _proj_body,
        out_shape=[jax.ShapeDtypeStruct((N, Hd), jnp.float32),
                   jax.ShapeDtypeStruct((G, 128), jnp.float32)],
    )(x, w, b, batch2d)


def _layer_body(h_ref, p_ref, eps_ref, w1_ref, b1_ref, w2_ref, b2_ref,
                batch_ref, hout_ref, pool_ref):
    agg = p_ref[0, :N, :] + p_ref[1, :N, :]
    z = (1.0 + eps_ref[...]) * h_ref[...] + agg
    z = jnp.dot(z, w1_ref[...], preferred_element_type=jnp.float32)
    z = jnp.maximum(z + b1_ref[...], 0.0)
    z = jnp.dot(z, w2_ref[...], preferred_element_type=jnp.float32)
    h2 = jnp.maximum(z + b2_ref[...], 0.0)
    hout_ref[...] = h2
    seg = lax.broadcasted_iota(jnp.int32, (G, N), 0)
    m = (batch_ref[...] == seg).astype(jnp.float32)
    pool_ref[...] = jnp.dot(m, h2, preferred_element_type=jnp.float32)


def _layer_call(h, p, eps2d, w1, b1, w2, b2, batch2d):
    return pl.pallas_call(
        _layer_body,
        out_shape=[jax.ShapeDtypeStruct((N, Hd), jnp.float32),
                   jax.ShapeDtypeStruct((G, Hd), jnp.float32)],
    )(h, p, eps2d, w1, b1, w2, b2, batch2d)


def _head_body(pool_ref, cnt_ref, jkw_ref, jkb_ref, w1_ref, b1_ref,
               w2_ref, b2_ref, w3_ref, b3_ref, out_ref):
    cnt = cnt_ref[:, 0:1]
    acc = jnp.zeros((G, Hd), jnp.float32)
    for i in range(L):
        hgi = jnp.dot(pool_ref[i], jkw_ref[i],
                      preferred_element_type=jnp.float32)
        hgi = hgi + cnt * jkb_ref[i]
        acc = acc + jnp.dot(hgi, w1_ref[i],
                            preferred_element_type=jnp.float32)
    hg = jnp.maximum(acc + b1_ref[...], 0.0)
    hg = jnp.dot(hg, w2_ref[...], preferred_element_type=jnp.float32)
    hg = jnp.maximum(hg + b2_ref[...], 0.0)
    out_ref[...] = (jnp.dot(hg, w3_ref[...],
                            preferred_element_type=jnp.float32)
                    + b3_ref[...])


def _head_call(pooled, cnt, jkw, jkb, w1c, b1, w2, b2, w3, b3):
    return pl.pallas_call(
        _head_body,
        out_shape=jax.ShapeDtypeStruct((G, T), jnp.float32),
    )(pooled, cnt, jkw, jkb, w1c, b1, w2, b2, w3, b3)


def _fold_bn(w, b, g, bb):
    s = g * _BN_S
    return w * s[None, :], (b * s + bb)[None, :]


def kernel(x, edge_index, batch, params):
    p = params
    src = edge_index[0]
    dst = edge_index[1]
    npad = EPAD - E
    pad_src = (jnp.arange(npad, dtype=jnp.int32) % N)
    pad_dst = N + (jnp.arange(npad, dtype=jnp.int32) % (N_ACC - N))
    src_p = jnp.concatenate([src, pad_src])
    dst_p = jnp.concatenate([dst, pad_dst])
    zrows = jnp.zeros((ZR, D), jnp.float32)
    batch2d = batch.reshape(1, N)

    inw, inb = _fold_bn(p['inW'], p['inb'], p['ing'], p['inbb'])
    h, cnt = _proj_call(x, inw, inb, batch2d)

    pooled = []
    for i in range(L):
        gl = p['gin'][i]
        w1, b1 = _fold_bn(gl['W1'], gl['b1'], gl['g1'], gl['bb1'])
        w2, b2 = _fold_bn(gl['W2'], gl['b2'], gl['g2'], gl['bb2'])
        eps2d = gl['eps'].reshape(1, 1)
        part = _aggregate(h, src_p, dst_p, zrows)
        h, pool_i = _layer_call(h, part, eps2d, w1, b1, w2, b2, batch2d)
        pooled.append(pool_i)
    pooled = jnp.stack(pooled)

    jkw = jnp.stack([p['jk'][i]['W'] for i in range(L)])
    jkb = jnp.stack([p['jk'][i]['b'].reshape(1, Hd) for i in range(L)])
    po = p['out']
    ow1, ob1 = _fold_bn(po['W1'], po['b1'], po['g1'], po['bb1'])
    ow2, ob2 = _fold_bn(po['W2'], po['b2'], po['g2'], po['bb2'])
    w1c = ow1.reshape(L, Hd, Hd)
    return _head_call(pooled, cnt, jkw, jkb, w1c, ob1,
                      ow2, ob2, po['W3'], po['b3'].reshape(1, T))


# speedup vs baseline: 12.9336x; 1.2001x over previous
import functools

import jax
import jax.numpy as jnp
import numpy as np
from jax import lax
from jax.experimental import pallas as pl
from jax.experimental.pallas import tpu as pltpu
from jax.experimental.pallas import tpu_sc as plsc

N = 10000
E = 320000
D = 128
Hd = 128
L = 5
G = 64
T = 12

NC = 2
NS = 16
NW = NC * NS
CH = 120
CPW = 84
EPAD = NW * CH * CPW
NB = 3
N_ACC = 10112
ZR = N_ACC // NS

_BN_S = 1.0 / np.sqrt(1.0 + 1e-5)


NQ = 4


def _agg_body(h_hbm, src_hbm, dst_hbm, zrows_hbm, out_hbm,
              src0, src1, src2, src3, dst0, dst1, dst2, dst3,
              rows0, rows1, rows2, acc,
              gsem0, gsem1, gsem2, ssem0, ssem1, ssem2,
              isem0, isem1, isem2, isem3):
    c = lax.axis_index("c")
    s = lax.axis_index("s")
    wid = s * NC + c
    ebase = wid * (CPW * CH)

    srcs = (src0, src1, src2, src3)
    dsts = (dst0, dst1, dst2, dst3)
    rows = (rows0, rows1, rows2)
    gsems = (gsem0, gsem1, gsem2)
    ssems = (ssem0, ssem1, ssem2)
    isems = (isem0, isem1, isem2, isem3)

    def load_idx(j, q):
        base = pl.multiple_of(ebase + j * CH, CH)
        pltpu.async_copy(src_hbm.at[pl.ds(base, CH)], srcs[q], isems[q])
        pltpu.async_copy(dst_hbm.at[pl.ds(base, CH)], dsts[q], isems[q])

    def wait_idx(q):
        pltpu.make_async_copy(src_hbm.at[pl.ds(0, CH)], srcs[q],
                              isems[q]).wait()
        pltpu.make_async_copy(dst_hbm.at[pl.ds(0, CH)], dsts[q],
                              isems[q]).wait()

    def gather(q, b):
        pltpu.async_copy(h_hbm.at[srcs[q]], rows[b], gsems[b])

    def wait_gather(b):
        pltpu.make_async_copy(h_hbm.at[pl.ds(0, CH)], rows[b],
                              gsems[b]).wait()

    def wait_scatter(b):
        pltpu.make_async_copy(h_hbm.at[pl.ds(0, CH)], rows[b],
                              ssems[b]).wait()

    pltpu.sync_copy(zrows_hbm, acc.at[pl.ds(s * ZR, ZR)])
    plsc.subcore_barrier()

    for q in range(NB):
        load_idx(q, q)
    for b in range(2):
        wait_idx(b)
        gather(b, b)

    def outer(i, carry):
        for bb in range(NB * NQ):
            j = i * (NB * NQ) + bb
            b = bb % NB
            b1 = (bb + 2) % NB
            q = bb % NQ
            wait_gather(b)
            pltpu.async_copy(rows[b], acc.at[dsts[q]], ssems[b], add=True)

            @pl.when(j > 0)
            def _():
                wait_scatter(b1)

            @pl.when(j + NB < CPW)
            def _():
                load_idx(j + NB, (q + NB) % NQ)

            @pl.when(j + 2 < CPW)
            def _():
                wait_idx((q + 2) % NQ)
                gather((q + 2) % NQ, b1)
        return carry

    lax.fori_loop(0, CPW // (NB * NQ), outer, 0)
    wait_scatter((CPW - 1) % NB)
    plsc.subcore_barrier()
    pltpu.sync_copy(acc.at[pl.ds(s * ZR, ZR)],
                    out_hbm.at[c, pl.ds(s * ZR, ZR)])


@functools.cache
def _make_aggregate():
    return pl.kernel(
        _agg_body,
        out_type=jax.ShapeDtypeStruct((NC, N_ACC, D), jnp.float32),
        mesh=plsc.VectorSubcoreMesh(core_axis_name="c", subcore_axis_name="s",
                                    num_cores=NC, num_subcores=NS),
        scratch_types=(
            [pltpu.VMEM((CH,), jnp.int32) for _ in range(2 * NQ)]
            + [pltpu.VMEM((CH, D), jnp.float32) for _ in range(NB)]
            + [pltpu.VMEM_SHARED((N_ACC, D), jnp.float32)]
            + [pltpu.SemaphoreType.DMA for _ in range(2 * NB + NQ)]
        ),
    )


def _aggregate(h, src_p, dst_p, zrows):
    return _make_aggregate()(h, src_p, dst_p, zrows)


def _proj_body(x_ref, w_ref, b_ref, batch_ref, h_ref, cnt_ref):
    h = jnp.dot(x_ref[...], w_ref[...], preferred_element_type=jnp.float32)
    h_ref[...] = jnp.maximum(h + b_ref[...], 0.0)
    seg = lax.broadcasted_iota(jnp.int32, (G, N), 0)
    m = (batch_ref[...] == seg).astype(jnp.float32)
    cnt_ref[...] = jnp.broadcast_to(jnp.sum(m, axis=1, keepdims=True),
                                    (G, 128))


def _proj_call(x, w, b, batch2d):
    return pl.pallas_call(
        _proj_body,
        out_shape=[jax.ShapeDtypeStruct((N, Hd), jnp.float32),
                   jax.ShapeDtypeStruct((G, 128), jnp.float32)],
    )(x, w, b, batch2d)


def _layer_body(h_ref, p_ref, eps_ref, w1_ref, b1_ref, w2_ref, b2_ref,
                batch_ref, hout_ref, pool_ref):
    agg = p_ref[0, :N, :] + p_ref[1, :N, :]
    z = (1.0 + eps_ref[...]) * h_ref[...] + agg
    z = jnp.dot(z, w1_ref[...], preferred_element_type=jnp.float32)
    z = jnp.maximum(z + b1_ref[...], 0.0)
    z = jnp.dot(z, w2_ref[...], preferred_element_type=jnp.float32)
    h2 = jnp.maximum(z + b2_ref[...], 0.0)
    hout_ref[...] = h2
    seg = lax.broadcasted_iota(jnp.int32, (G, N), 0)
    m = (batch_ref[...] == seg).astype(jnp.float32)
    pool_ref[...] = jnp.dot(m, h2, preferred_element_type=jnp.float32)


def _layer_call(h, p, eps2d, w1, b1, w2, b2, batch2d):
    return pl.pallas_call(
        _layer_body,
        out_shape=[jax.ShapeDtypeStruct((N, Hd), jnp.float32),
                   jax.ShapeDtypeStruct((G, Hd), jnp.float32)],
    )(h, p, eps2d, w1, b1, w2, b2, batch2d)


def _head_body(pool_ref, cnt_ref, jkw_ref, jkb_ref, w1_ref, b1_ref,
               w2_ref, b2_ref, w3_ref, b3_ref, out_ref):
    cnt = cnt_ref[:, 0:1]
    acc = jnp.zeros((G, Hd), jnp.float32)
    for i in range(L):
        hgi = jnp.dot(pool_ref[i], jkw_ref[i],
                      preferred_element_type=jnp.float32)
        hgi = hgi + cnt * jkb_ref[i]
        acc = acc + jnp.dot(hgi, w1_ref[i],
                            preferred_element_type=jnp.float32)
    hg = jnp.maximum(acc + b1_ref[...], 0.0)
    hg = jnp.dot(hg, w2_ref[...], preferred_element_type=jnp.float32)
    hg = jnp.maximum(hg + b2_ref[...], 0.0)
    out_ref[...] = (jnp.dot(hg, w3_ref[...],
                            preferred_element_type=jnp.float32)
                    + b3_ref[...])


def _head_call(pooled, cnt, jkw, jkb, w1c, b1, w2, b2, w3, b3):
    return pl.pallas_call(
        _head_body,
        out_shape=jax.ShapeDtypeStruct((G, T), jnp.float32),
    )(pooled, cnt, jkw, jkb, w1c, b1, w2, b2, w3, b3)


def _fold_bn(w, b, g, bb):
    s = g * _BN_S
    return w * s[None, :], (b * s + bb)[None, :]


def kernel(x, edge_index, batch, params):
    p = params
    src = edge_index[0]
    dst = edge_index[1]
    npad = EPAD - E
    pad_src = (jnp.arange(npad, dtype=jnp.int32) % N)
    pad_dst = N + (jnp.arange(npad, dtype=jnp.int32) % (N_ACC - N))
    src_p = jnp.concatenate([src, pad_src])
    dst_p = jnp.concatenate([dst, pad_dst])
    zrows = jnp.zeros((ZR, D), jnp.float32)
    batch2d = batch.reshape(1, N)

    inw, inb = _fold_bn(p['inW'], p['inb'], p['ing'], p['inbb'])
    h, cnt = _proj_call(x, inw, inb, batch2d)

    pooled = []
    for i in range(L):
        gl = p['gin'][i]
        w1, b1 = _fold_bn(gl['W1'], gl['b1'], gl['g1'], gl['bb1'])
        w2, b2 = _fold_bn(gl['W2'], gl['b2'], gl['g2'], gl['bb2'])
        eps2d = gl['eps'].reshape(1, 1)
        part = _aggregate(h, src_p, dst_p, zrows)
        h, pool_i = _layer_call(h, part, eps2d, w1, b1, w2, b2, batch2d)
        pooled.append(pool_i)
    pooled = jnp.stack(pooled)

    jkw = jnp.stack([p['jk'][i]['W'] for i in range(L)])
    jkb = jnp.stack([p['jk'][i]['b'].reshape(1, Hd) for i in range(L)])
    po = p['out']
    ow1, ob1 = _fold_bn(po['W1'], po['b1'], po['g1'], po['bb1'])
    ow2, ob2 = _fold_bn(po['W2'], po['b2'], po['g2'], po['bb2'])
    w1c = ow1.reshape(L, Hd, Hd)
    return _head_call(pooled, cnt, jkw, jkb, w1c, ob1,
                      ow2, ob2, po['W3'], po['b3'].reshape(1, T))


# speedup vs baseline: 12.9394x; 1.0005x over previous
import functools

import jax
import jax.numpy as jnp
import numpy as np
from jax import lax
from jax.experimental import pallas as pl
from jax.experimental.pallas import tpu as pltpu
from jax.experimental.pallas import tpu_sc as plsc

N = 10000
E = 320000
D = 128
Hd = 128
L = 5
G = 64
T = 12

NC = 2
NS = 16
NW = NC * NS
CH = 120
CPW = 84
EPAD = NW * CH * CPW
NB = 3
N_ACC = 10112
ZR = N_ACC // NS

_BN_S = 1.0 / np.sqrt(1.0 + 1e-5)


NQ = 4


def _agg_body(h_hbm, src_hbm, dst_hbm, zrows_hbm, out_hbm,
              src0, src1, src2, src3, dst0, dst1, dst2, dst3,
              rows0, rows1, rows2, acc,
              gsem0, gsem1, gsem2, ssem0, ssem1, ssem2,
              isem0, isem1, isem2, isem3):
    c = lax.axis_index("c")
    s = lax.axis_index("s")
    wid = s * NC + c
    ebase = wid * (CPW * CH)

    srcs = (src0, src1, src2, src3)
    dsts = (dst0, dst1, dst2, dst3)
    rows = (rows0, rows1, rows2)
    gsems = (gsem0, gsem1, gsem2)
    ssems = (ssem0, ssem1, ssem2)
    isems = (isem0, isem1, isem2, isem3)

    def load_idx(j, q):
        base = pl.multiple_of(ebase + j * CH, CH)
        pltpu.async_copy(src_hbm.at[pl.ds(base, CH)], srcs[q], isems[q])
        pltpu.async_copy(dst_hbm.at[pl.ds(base, CH)], dsts[q], isems[q])

    def wait_idx(q):
        pltpu.make_async_copy(src_hbm.at[pl.ds(0, CH)], srcs[q],
                              isems[q]).wait()
        pltpu.make_async_copy(dst_hbm.at[pl.ds(0, CH)], dsts[q],
                              isems[q]).wait()

    CH2 = 64

    def gather(q, b):
        pltpu.async_copy(h_hbm.at[srcs[q].at[pl.ds(0, CH2)]],
                         rows[b].at[pl.ds(0, CH2)], gsems[b])
        pltpu.async_copy(h_hbm.at[srcs[q].at[pl.ds(CH2, CH - CH2)]],
                         rows[b].at[pl.ds(CH2, CH - CH2)], gsems[b])

    def wait_gather(b):
        pltpu.make_async_copy(h_hbm.at[pl.ds(0, CH2)],
                              rows[b].at[pl.ds(0, CH2)], gsems[b]).wait()
        pltpu.make_async_copy(h_hbm.at[pl.ds(0, CH - CH2)],
                              rows[b].at[pl.ds(CH2, CH - CH2)],
                              gsems[b]).wait()

    def wait_scatter(b):
        pltpu.make_async_copy(h_hbm.at[pl.ds(0, CH)], rows[b],
                              ssems[b]).wait()

    pltpu.sync_copy(zrows_hbm, acc.at[pl.ds(s * ZR, ZR)])
    plsc.subcore_barrier()

    for q in range(NB):
        load_idx(q, q)
    for b in range(2):
        wait_idx(b)
        gather(b, b)

    def outer(i, carry):
        for bb in range(NB * NQ):
            j = i * (NB * NQ) + bb
            b = bb % NB
            b1 = (bb + 2) % NB
            q = bb % NQ
            wait_gather(b)
            pltpu.async_copy(rows[b], acc.at[dsts[q]], ssems[b], add=True)

            @pl.when(j > 0)
            def _():
                wait_scatter(b1)

            @pl.when(j + NB < CPW)
            def _():
                load_idx(j + NB, (q + NB) % NQ)

            @pl.when(j + 2 < CPW)
            def _():
                wait_idx((q + 2) % NQ)
                gather((q + 2) % NQ, b1)
        return carry

    lax.fori_loop(0, CPW // (NB * NQ), outer, 0)
    wait_scatter((CPW - 1) % NB)
    plsc.subcore_barrier()
    pltpu.sync_copy(acc.at[pl.ds(s * ZR, ZR)],
                    out_hbm.at[c, pl.ds(s * ZR, ZR)])


@functools.cache
def _make_aggregate():
    return pl.kernel(
        _agg_body,
        out_type=jax.ShapeDtypeStruct((NC, N_ACC, D), jnp.float32),
        mesh=plsc.VectorSubcoreMesh(core_axis_name="c", subcore_axis_name="s",
                                    num_cores=NC, num_subcores=NS),
        scratch_types=(
            [pltpu.VMEM((CH,), jnp.int32) for _ in range(2 * NQ)]
            + [pltpu.VMEM((CH, D), jnp.float32) for _ in range(NB)]
            + [pltpu.VMEM_SHARED((N_ACC, D), jnp.float32)]
            + [pltpu.SemaphoreType.DMA for _ in range(2 * NB + NQ)]
        ),
    )


def _aggregate(h, src_p, dst_p, zrows):
    return _make_aggregate()(h, src_p, dst_p, zrows)


def _proj_body(x_ref, w_ref, b_ref, batch_ref, h_ref, cnt_ref):
    h = jnp.dot(x_ref[...], w_ref[...], preferred_element_type=jnp.float32)
    h_ref[...] = jnp.maximum(h + b_ref[...], 0.0)
    seg = lax.broadcasted_iota(jnp.int32, (G, N), 0)
    m = (batch_ref[...] == seg).astype(jnp.float32)
    cnt_ref[...] = jnp.broadcast_to(jnp.sum(m, axis=1, keepdims=True),
                                    (G, 128))


def _proj_call(x, w, b, batch2d):
    return pl.pallas_call(
        _proj_body,
        out_shape=[jax.ShapeDtypeStruct((N, Hd), jnp.float32),
                   jax.ShapeDtypeStruct((G, 128), jnp.float32)],
    )(x, w, b, batch2d)


def _layer_body(h_ref, p_ref, eps_ref, w1_ref, b1_ref, w2_ref, b2_ref,
                batch_ref, hout_ref, pool_ref):
    agg = p_ref[0, :N, :] + p_ref[1, :N, :]
    z = (1.0 + eps_ref[...]) * h_ref[...] + agg
    z = jnp.dot(z, w1_ref[...], preferred_element_type=jnp.float32)
    z = jnp.maximum(z + b1_ref[...], 0.0)
    z = jnp.dot(z, w2_ref[...], preferred_element_type=jnp.float32)
    h2 = jnp.maximum(z + b2_ref[...], 0.0)
    hout_ref[...] = h2
    seg = lax.broadcasted_iota(jnp.int32, (G, N), 0)
    m = (batch_ref[...] == seg).astype(jnp.float32)
    pool_ref[...] = jnp.dot(m, h2, preferred_element_type=jnp.float32)


def _layer_call(h, p, eps2d, w1, b1, w2, b2, batch2d):
    return pl.pallas_call(
        _layer_body,
        out_shape=[jax.ShapeDtypeStruct((N, Hd), jnp.float32),
                   jax.ShapeDtypeStruct((G, Hd), jnp.float32)],
    )(h, p, eps2d, w1, b1, w2, b2, batch2d)


def _head_body(pool_ref, cnt_ref, jkw_ref, jkb_ref, w1_ref, b1_ref,
               w2_ref, b2_ref, w3_ref, b3_ref, out_ref):
    cnt = cnt_ref[:, 0:1]
    acc = jnp.zeros((G, Hd), jnp.float32)
    for i in range(L):
        hgi = jnp.dot(pool_ref[i], jkw_ref[i],
                      preferred_element_type=jnp.float32)
        hgi = hgi + cnt * jkb_ref[i]
        acc = acc + jnp.dot(hgi, w1_ref[i],
                            preferred_element_type=jnp.float32)
    hg = jnp.maximum(acc + b1_ref[...], 0.0)
    hg = jnp.dot(hg, w2_ref[...], preferred_element_type=jnp.float32)
    hg = jnp.maximum(hg + b2_ref[...], 0.0)
    out_ref[...] = (jnp.dot(hg, w3_ref[...],
                            preferred_element_type=jnp.float32)
                    + b3_ref[...])


def _head_call(pooled, cnt, jkw, jkb, w1c, b1, w2, b2, w3, b3):
    return pl.pallas_call(
        _head_body,
        out_shape=jax.ShapeDtypeStruct((G, T), jnp.float32),
    )(pooled, cnt, jkw, jkb, w1c, b1, w2, b2, w3, b3)


def _fold_bn(w, b, g, bb):
    s = g * _BN_S
    return w * s[None, :], (b * s + bb)[None, :]


def kernel(x, edge_index, batch, params):
    p = params
    src = edge_index[0]
    dst = edge_index[1]
    npad = EPAD - E
    pad_src = (jnp.arange(npad, dtype=jnp.int32) % N)
    pad_dst = N + (jnp.arange(npad, dtype=jnp.int32) % (N_ACC - N))
    src_p = jnp.concatenate([src, pad_src])
    dst_p = jnp.concatenate([dst, pad_dst])
    zrows = jnp.zeros((ZR, D), jnp.float32)
    batch2d = batch.reshape(1, N)

    inw, inb = _fold_bn(p['inW'], p['inb'], p['ing'], p['inbb'])
    h, cnt = _proj_call(x, inw, inb, batch2d)

    pooled = []
    for i in range(L):
        gl = p['gin'][i]
        w1, b1 = _fold_bn(gl['W1'], gl['b1'], gl['g1'], gl['bb1'])
        w2, b2 = _fold_bn(gl['W2'], gl['b2'], gl['g2'], gl['bb2'])
        eps2d = gl['eps'].reshape(1, 1)
        part = _aggregate(h, src_p, dst_p, zrows)
        h, pool_i = _layer_call(h, part, eps2d, w1, b1, w2, b2, batch2d)
        pooled.append(pool_i)
    pooled = jnp.stack(pooled)

    jkw = jnp.stack([p['jk'][i]['W'] for i in range(L)])
    jkb = jnp.stack([p['jk'][i]['b'].reshape(1, Hd) for i in range(L)])
    po = p['out']
    ow1, ob1 = _fold_bn(po['W1'], po['b1'], po['g1'], po['bb1'])
    ow2, ob2 = _fold_bn(po['W2'], po['b2'], po['g2'], po['bb2'])
    w1c = ow1.reshape(L, Hd, Hd)
    return _head_call(pooled, cnt, jkw, jkb, w1c, ob1,
                      ow2, ob2, po['W3'], po['b3'].reshape(1, T))


# speedup vs baseline: 13.0729x; 1.0103x over previous
import functools

import jax
import jax.numpy as jnp
import numpy as np
from jax import lax
from jax.experimental import pallas as pl
from jax.experimental.pallas import tpu as pltpu
from jax.experimental.pallas import tpu_sc as plsc

N = 10000
E = 320000
D = 128
Hd = 128
L = 5
G = 64
T = 12

NC = 2
NS = 16
NW = NC * NS
CH = 120
CPW = 84
EPAD = NW * CH * CPW
NB = 3
N_ACC = 10112
ZR = N_ACC // NS

_BN_S = 1.0 / np.sqrt(1.0 + 1e-5)


NQ = 4


def _agg_body(h_hbm, src_hbm, dst_hbm, zrows_hbm, out_hbm,
              src0, src1, src2, src3, dst0, dst1, dst2, dst3,
              rows0, rows1, rows2, acc,
              gsem0, gsem1, gsem2, ssem0, ssem1, ssem2,
              isem0, isem1, isem2, isem3):
    c = lax.axis_index("c")
    s = lax.axis_index("s")
    wid = s * NC + c
    ebase = wid * (CPW * CH)

    srcs = (src0, src1, src2, src3)
    dsts = (dst0, dst1, dst2, dst3)
    rows = (rows0, rows1, rows2)
    gsems = (gsem0, gsem1, gsem2)
    ssems = (ssem0, ssem1, ssem2)
    isems = (isem0, isem1, isem2, isem3)

    def load_idx(j, q):
        base = pl.multiple_of(ebase + j * CH, CH)
        pltpu.async_copy(src_hbm.at[pl.ds(base, CH)], srcs[q], isems[q])
        pltpu.async_copy(dst_hbm.at[pl.ds(base, CH)], dsts[q], isems[q])

    def wait_idx(q):
        pltpu.make_async_copy(src_hbm.at[pl.ds(0, CH)], srcs[q],
                              isems[q]).wait()
        pltpu.make_async_copy(dst_hbm.at[pl.ds(0, CH)], dsts[q],
                              isems[q]).wait()

    def gather(q, b):
        pltpu.async_copy(h_hbm.at[srcs[q]], rows[b], gsems[b])

    def wait_gather(b):
        pltpu.make_async_copy(h_hbm.at[pl.ds(0, CH)], rows[b],
                              gsems[b]).wait()

    def wait_scatter(b):
        pltpu.make_async_copy(h_hbm.at[pl.ds(0, CH)], rows[b],
                              ssems[b]).wait()

    for q in range(NB):
        load_idx(q, q)
    for b in range(2):
        wait_idx(b)
        gather(b, b)

    pltpu.sync_copy(zrows_hbm, acc.at[pl.ds(s * ZR, ZR)])
    plsc.subcore_barrier()

    def outer(i, carry):
        for bb in range(NB * NQ):
            j = i * (NB * NQ) + bb
            b = bb % NB
            b1 = (bb + 2) % NB
            q = bb % NQ
            wait_gather(b)
            pltpu.async_copy(rows[b], acc.at[dsts[q]], ssems[b], add=True)

            @pl.when(j > 0)
            def _():
                wait_scatter(b1)

            @pl.when(j + NB < CPW)
            def _():
                load_idx(j + NB, (q + NB) % NQ)

            @pl.when(j + 2 < CPW)
            def _():
                wait_idx((q + 2) % NQ)
                gather((q + 2) % NQ, b1)
        return carry

    lax.fori_loop(0, CPW // (NB * NQ), outer, 0)
    wait_scatter((CPW - 1) % NB)
    plsc.subcore_barrier()
    pltpu.sync_copy(acc.at[pl.ds(s * ZR, ZR)],
                    out_hbm.at[c, pl.ds(s * ZR, ZR)])


@functools.cache
def _make_aggregate():
    return pl.kernel(
        _agg_body,
        out_type=jax.ShapeDtypeStruct((NC, N_ACC, D), jnp.float32),
        mesh=plsc.VectorSubcoreMesh(core_axis_name="c", subcore_axis_name="s",
                                    num_cores=NC, num_subcores=NS),
        scratch_types=(
            [pltpu.VMEM((CH,), jnp.int32) for _ in range(2 * NQ)]
            + [pltpu.VMEM((CH, D), jnp.float32) for _ in range(NB)]
            + [pltpu.VMEM_SHARED((N_ACC, D), jnp.float32)]
            + [pltpu.SemaphoreType.DMA for _ in range(2 * NB + NQ)]
        ),
    )


def _aggregate(h, src_p, dst_p, zrows):
    return _make_aggregate()(h, src_p, dst_p, zrows)


def _proj_body(x_ref, w_ref, b_ref, batch_ref, h_ref, cnt_ref):
    h = jnp.dot(x_ref[...], w_ref[...], preferred_element_type=jnp.float32)
    h_ref[...] = jnp.maximum(h + b_ref[...], 0.0)
    seg = lax.broadcasted_iota(jnp.int32, (G, N), 0)
    m = (batch_ref[...] == seg).astype(jnp.float32)
    cnt_ref[...] = jnp.broadcast_to(jnp.sum(m, axis=1, keepdims=True),
                                    (G, 128))


def _proj_call(x, w, b, batch2d):
    return pl.pallas_call(
        _proj_body,
        out_shape=[jax.ShapeDtypeStruct((N, Hd), jnp.float32),
                   jax.ShapeDtypeStruct((G, 128), jnp.float32)],
    )(x, w, b, batch2d)


def _layer_body(h_ref, p_ref, eps_ref, w1_ref, b1_ref, w2_ref, b2_ref,
                batch_ref, hout_ref, pool_ref):
    agg = p_ref[0, :N, :] + p_ref[1, :N, :]
    z = (1.0 + eps_ref[...]) * h_ref[...] + agg
    z = jnp.dot(z, w1_ref[...], preferred_element_type=jnp.float32)
    z = jnp.maximum(z + b1_ref[...], 0.0)
    z = jnp.dot(z, w2_ref[...], preferred_element_type=jnp.float32)
    h2 = jnp.maximum(z + b2_ref[...], 0.0)
    hout_ref[...] = h2
    seg = lax.broadcasted_iota(jnp.int32, (G, N), 0)
    m = (batch_ref[...] == seg).astype(jnp.float32)
    pool_ref[...] = jnp.dot(m, h2, preferred_element_type=jnp.float32)


def _layer_call(h, p, eps2d, w1, b1, w2, b2, batch2d):
    return pl.pallas_call(
        _layer_body,
        out_shape=[jax.ShapeDtypeStruct((N, Hd), jnp.float32),
                   jax.ShapeDtypeStruct((G, Hd), jnp.float32)],
    )(h, p, eps2d, w1, b1, w2, b2, batch2d)


def _head_body(pool_ref, cnt_ref, jkw_ref, jkb_ref, w1_ref, b1_ref,
               w2_ref, b2_ref, w3_ref, b3_ref, out_ref):
    cnt = cnt_ref[:, 0:1]
    acc = jnp.zeros((G, Hd), jnp.float32)
    for i in range(L):
        hgi = jnp.dot(pool_ref[i], jkw_ref[i],
                      preferred_element_type=jnp.float32)
        hgi = hgi + cnt * jkb_ref[i]
        acc = acc + jnp.dot(hgi, w1_ref[i],
                            preferred_element_type=jnp.float32)
    hg = jnp.maximum(acc + b1_ref[...], 0.0)
    hg = jnp.dot(hg, w2_ref[...], preferred_element_type=jnp.float32)
    hg = jnp.maximum(hg + b2_ref[...], 0.0)
    out_ref[...] = (jnp.dot(hg, w3_ref[...],
                            preferred_element_type=jnp.float32)
                    + b3_ref[...])


def _head_call(pooled, cnt, jkw, jkb, w1c, b1, w2, b2, w3, b3):
    return pl.pallas_call(
        _head_body,
        out_shape=jax.ShapeDtypeStruct((G, T), jnp.float32),
    )(pooled, cnt, jkw, jkb, w1c, b1, w2, b2, w3, b3)


def _fold_bn(w, b, g, bb):
    s = g * _BN_S
    return w * s[None, :], (b * s + bb)[None, :]


def kernel(x, edge_index, batch, params):
    p = params
    src = edge_index[0]
    dst = edge_index[1]
    npad = EPAD - E
    pad_src = (jnp.arange(npad, dtype=jnp.int32) % N)
    pad_dst = N + (jnp.arange(npad, dtype=jnp.int32) % (N_ACC - N))
    src_p = jnp.concatenate([src, pad_src])
    dst_p = jnp.concatenate([dst, pad_dst])
    zrows = jnp.zeros((ZR, D), jnp.float32)
    batch2d = batch.reshape(1, N)

    inw, inb = _fold_bn(p['inW'], p['inb'], p['ing'], p['inbb'])
    h, cnt = _proj_call(x, inw, inb, batch2d)

    pooled = []
    for i in range(L):
        gl = p['gin'][i]
        w1, b1 = _fold_bn(gl['W1'], gl['b1'], gl['g1'], gl['bb1'])
        w2, b2 = _fold_bn(gl['W2'], gl['b2'], gl['g2'], gl['bb2'])
        eps2d = gl['eps'].reshape(1, 1)
        part = _aggregate(h, src_p, dst_p, zrows)
        h, pool_i = _layer_call(h, part, eps2d, w1, b1, w2, b2, batch2d)
        pooled.append(pool_i)
    pooled = jnp.stack(pooled)

    jkw = jnp.stack([p['jk'][i]['W'] for i in range(L)])
    jkb = jnp.stack([p['jk'][i]['b'].reshape(1, Hd) for i in range(L)])
    po = p['out']
    ow1, ob1 = _fold_bn(po['W1'], po['b1'], po['g1'], po['bb1'])
    ow2, ob2 = _fold_bn(po['W2'], po['b2'], po['g2'], po['bb2'])
    w1c = ow1.reshape(L, Hd, Hd)
    return _head_call(pooled, cnt, jkw, jkb, w1c, ob1,
                      ow2, ob2, po['W3'], po['b3'].reshape(1, T))
